# baseline (speedup 1.0000x reference)
import functools

import jax
import jax.numpy as jnp
from jax import lax
from jax.experimental import pallas as pl
from jax.experimental.pallas import tpu as pltpu
from jax.experimental.pallas import tpu_sc as plsc

D = 768
B = 4
T = 2048
N = B * T
EPS = 1e-6
NC, NS, L = 2, 16, 16
NW = NC * NS
PER_W = N // NW
C = 64
NCHUNK = PER_W // C
NJ = D // L


def _rsqrt_scalar(a):
    i = lax.bitcast_convert_type(a, jnp.int32)
    i = jnp.int32(0x5F3759DF) - (i >> 1)
    y = lax.bitcast_convert_type(i, jnp.float32)
    half_a = 0.5 * a
    for _ in range(3):
        y = y * (1.5 - half_a * y * y)
    return y


_mesh = plsc.VectorSubcoreMesh(core_axis_name="c", subcore_axis_name="s")


@functools.partial(
    pl.kernel,
    mesh=_mesh,
    out_type=jax.ShapeDtypeStruct((N, D), jnp.float32),
    scratch_types=[
        pltpu.VMEM((C,), jnp.int32),
        pltpu.VMEM((C, D), jnp.float32),
        pltpu.VMEM((C, D), jnp.float32),
        pltpu.VMEM((D,), jnp.float32),
        pltpu.SemaphoreType.DMA,
    ],
)
def _emb_kernel(idx_hbm, ww_hbm, wp_hbm, sc_hbm, out_hbm,
                idx_v, tok_v, pos_v, scale_v, sem):
    wid = lax.axis_index("s") * NC + lax.axis_index("c")
    base = wid * PER_W
    pltpu.sync_copy(sc_hbm, scale_v)

    def chunk_body(c, carry):
        row0 = base + c * C
        p0 = lax.rem(row0, T)
        pltpu.sync_copy(idx_hbm.at[pl.ds(row0, C)], idx_v)
        gather = pltpu.async_copy(ww_hbm.at[idx_v], tok_v, sem)
        pltpu.sync_copy(wp_hbm.at[pl.ds(p0, C)], pos_v)
        gather.wait()

        def row_body(r, cc):
            acc = jnp.zeros((L,), jnp.float32)
            for j in range(NJ):
                sl = pl.ds(j * L, L)
                x = tok_v[r, sl] + pos_v[r, sl]
                tok_v[r, sl] = x
                acc = acc + x * x
            s = acc[0]
            for l in range(1, L):
                s = s + acc[l]
            ms = s * (1.0 / D) + EPS
            rstd = jnp.full((L,), _rsqrt_scalar(ms), jnp.float32)
            for j in range(NJ):
                sl = pl.ds(j * L, L)
                tok_v[r, sl] = tok_v[r, sl] * rstd * scale_v[sl]
            return cc

        lax.fori_loop(0, C, row_body, 0)
        pltpu.sync_copy(tok_v, out_hbm.at[pl.ds(row0, C)])
        return carry

    lax.fori_loop(0, NCHUNK, chunk_body, 0)


def kernel(idx, W_word, W_pos, rms_scale):
    out = _emb_kernel(idx.reshape(N), W_word, W_pos, rms_scale)
    return out.reshape(B, T, D)


# speedup vs baseline: 1.3813x; 1.3813x over previous
import functools

import jax
import jax.numpy as jnp
from jax import lax
from jax.experimental import pallas as pl
from jax.experimental.pallas import tpu as pltpu
from jax.experimental.pallas import tpu_sc as plsc

D = 768
B = 4
T = 2048
N = B * T
EPS = 1e-6
NC, NS, L = 2, 16, 16
NW = NC * NS
TW = T // NW
C = 32
NCHUNK = (B * TW) // C
NJ = D // L


def _rsqrt_scalar(a):
    i = lax.bitcast_convert_type(a, jnp.int32)
    i = jnp.int32(0x5F3759DF) - (i >> 1)
    y = lax.bitcast_convert_type(i, jnp.float32)
    half_a = 0.5 * a
    for _ in range(3):
        y = y * (1.5 - half_a * y * y)
    return y


_mesh = plsc.VectorSubcoreMesh(core_axis_name="c", subcore_axis_name="s")


@functools.partial(
    pl.kernel,
    mesh=_mesh,
    out_type=jax.ShapeDtypeStruct((N, D), jnp.float32),
    scratch_types=[
        pltpu.VMEM((2, C), jnp.int32),
        pltpu.VMEM((C, D), jnp.float32),
        pltpu.VMEM((C, D), jnp.float32),
        pltpu.VMEM((TW, D), jnp.float32),
        pltpu.VMEM((D,), jnp.float32),
        pltpu.SemaphoreType.DMA,
        pltpu.SemaphoreType.DMA,
        pltpu.SemaphoreType.DMA,
        pltpu.SemaphoreType.DMA,
    ],
)
def _emb_kernel(idx_hbm, ww_hbm, wp_hbm, sc_hbm, out_hbm,
                idx_v, tok0_v, tok1_v, pos_v, scale_v,
                g0_sem, g1_sem, o0_sem, o1_sem):
    wid = lax.axis_index("s") * NC + lax.axis_index("c")
    t0 = wid * TW
    toks = (tok0_v, tok1_v)
    gsems = (g0_sem, g1_sem)
    osems = (o0_sem, o1_sem)

    pltpu.sync_copy(sc_hbm, scale_v)
    pltpu.sync_copy(wp_hbm.at[pl.ds(t0, TW)], pos_v)
    scale_regs = [scale_v[pl.ds(j * L, L)] for j in range(NJ)]

    def flat0(c):
        return (c >> 1) * T + t0 + (c & 1) * C

    def start_gather(c):
        pltpu.sync_copy(idx_hbm.at[pl.ds(flat0(c), C)], idx_v.at[c % 2])
        return pltpu.async_copy(ww_hbm.at[idx_v.at[c % 2]], toks[c % 2],
                                gsems[c % 2])

    def compute(c):
        tok_v = toks[c % 2]
        poff = (c & 1) * C

        def row_body(r, cc):
            acc = jnp.zeros((L,), jnp.float32)
            for j in range(NJ):
                sl = pl.ds(j * L, L)
                x = tok_v[r, sl] + pos_v[poff + r, sl]
                tok_v[r, sl] = x
                acc = acc + x * x
            s = acc[0]
            for l in range(1, L):
                s = s + acc[l]
            ms = s * (1.0 / D) + EPS
            rstd = jnp.full((L,), _rsqrt_scalar(ms), jnp.float32)
            for j in range(NJ):
                sl = pl.ds(j * L, L)
                tok_v[r, sl] = tok_v[r, sl] * rstd * scale_regs[j]
            return cc

        lax.fori_loop(0, C, row_body, 0)

    gathers = {0: start_gather(0)}
    outs = {}
    for c in range(NCHUNK):
        if c + 1 < NCHUNK:
            if c - 1 in outs:
                outs[c - 1].wait()
            gathers[c + 1] = start_gather(c + 1)
        gathers[c].wait()
        compute(c)
        outs[c] = pltpu.async_copy(toks[c % 2],
                                   out_hbm.at[pl.ds(flat0(c), C)],
                                   osems[c % 2])
    outs[NCHUNK - 2].wait()
    outs[NCHUNK - 1].wait()


def kernel(idx, W_word, W_pos, rms_scale):
    out = _emb_kernel(idx.reshape(N), W_word, W_pos, rms_scale)
    return out.reshape(B, T, D)


# speedup vs baseline: 1.4055x; 1.0175x over previous
import functools

import jax
import jax.numpy as jnp
from jax import lax
from jax.experimental import pallas as pl
from jax.experimental.pallas import tpu as pltpu
from jax.experimental.pallas import tpu_sc as plsc

D = 768
B = 4
T = 2048
N = B * T
EPS = 1e-6
NC, NS, L = 2, 16, 16
NW = NC * NS
TW = T // NW
C = 32
NCHUNK = (B * TW) // C
NJ = D // L


def _rsqrt_scalar(a):
    i = lax.bitcast_convert_type(a, jnp.int32)
    i = jnp.int32(0x5F3759DF) - (i >> 1)
    y = lax.bitcast_convert_type(i, jnp.float32)
    half_a = 0.5 * a
    for _ in range(2):
        y = y * (1.5 - half_a * y * y)
    return y


_mesh = plsc.VectorSubcoreMesh(core_axis_name="c", subcore_axis_name="s")


@functools.partial(
    pl.kernel,
    mesh=_mesh,
    out_type=jax.ShapeDtypeStruct((N, D), jnp.float32),
    scratch_types=[
        pltpu.VMEM((2, C), jnp.int32),
        pltpu.VMEM((C, D), jnp.float32),
        pltpu.VMEM((C, D), jnp.float32),
        pltpu.VMEM((TW, D), jnp.float32),
        pltpu.VMEM((D,), jnp.float32),
        pltpu.SemaphoreType.DMA,
        pltpu.SemaphoreType.DMA,
        pltpu.SemaphoreType.DMA,
        pltpu.SemaphoreType.DMA,
    ],
)
def _emb_kernel(idx_hbm, ww_hbm, wp_hbm, sc_hbm, out_hbm,
                idx_v, tok0_v, tok1_v, pos_v, scale_v,
                g0_sem, g1_sem, o0_sem, o1_sem):
    wid = lax.axis_index("s") * NC + lax.axis_index("c")
    t0 = wid * TW
    toks = (tok0_v, tok1_v)
    gsems = (g0_sem, g1_sem)
    osems = (o0_sem, o1_sem)

    pltpu.sync_copy(sc_hbm, scale_v)
    pltpu.sync_copy(wp_hbm.at[pl.ds(t0, TW)], pos_v)
    scale_regs = [scale_v[pl.ds(j * L, L)] for j in range(NJ)]

    def flat0(c):
        return (c >> 1) * T + t0 + (c & 1) * C

    def start_gather(c):
        pltpu.sync_copy(idx_hbm.at[pl.ds(flat0(c), C)], idx_v.at[c % 2])
        return pltpu.async_copy(ww_hbm.at[idx_v.at[c % 2]], toks[c % 2],
                                gsems[c % 2])

    def compute(c):
        tok_v = toks[c % 2]
        poff = (c & 1) * C

        def row_body(r, cc):
            accs = [jnp.zeros((L,), jnp.float32) for _ in range(4)]
            for j in range(NJ):
                sl = pl.ds(j * L, L)
                x = tok_v[r, sl] + pos_v[poff + r, sl]
                tok_v[r, sl] = x
                accs[j % 4] = accs[j % 4] + x * x
            acc = (accs[0] + accs[1]) + (accs[2] + accs[3])
            lanes = [acc[l] for l in range(L)]
            while len(lanes) > 1:
                lanes = [lanes[i] + lanes[i + 1]
                         for i in range(0, len(lanes), 2)]
            ms = lanes[0] * (1.0 / D) + EPS
            rstd = jnp.full((L,), _rsqrt_scalar(ms), jnp.float32)
            for j in range(NJ):
                sl = pl.ds(j * L, L)
                tok_v[r, sl] = tok_v[r, sl] * rstd * scale_regs[j]
            return cc

        lax.fori_loop(0, C, row_body, 0)

    gathers = {0: start_gather(0)}
    outs = {}
    for c in range(NCHUNK):
        if c + 1 < NCHUNK:
            if c - 1 in outs:
                outs[c - 1].wait()
            gathers[c + 1] = start_gather(c + 1)
        gathers[c].wait()
        compute(c)
        outs[c] = pltpu.async_copy(toks[c % 2],
                                   out_hbm.at[pl.ds(flat0(c), C)],
                                   osems[c % 2])
    outs[NCHUNK - 2].wait()
    outs[NCHUNK - 1].wait()


def kernel(idx, W_word, W_pos, rms_scale):
    out = _emb_kernel(idx.reshape(N), W_word, W_pos, rms_scale)
    return out.reshape(B, T, D)
